# baseline (speedup 1.0000x reference)
import functools

import jax
import jax.numpy as jnp
from jax import lax
from jax.experimental import pallas as pl
from jax.experimental.pallas import tpu as pltpu
from jax.experimental.pallas import tpu_sc as plsc

B, N, D = 4, 8192, 1024

_info = plsc.get_sparse_core_info()
NC, NS = _info.num_cores, _info.num_subcores
NW = NC * NS
ROWS_PER_W = N // NW
CHUNK = 32
NCHUNK = ROWS_PER_W // CHUNK
NBUF = 2

_mesh = plsc.VectorSubcoreMesh(core_axis_name="c", subcore_axis_name="s")


@functools.partial(
    pl.kernel,
    mesh=_mesh,
    out_type=jax.ShapeDtypeStruct((B, N, D), jnp.float32),
    scratch_types=[
        pltpu.VMEM((NBUF, CHUNK, D), jnp.float32),
        pltpu.SemaphoreType.DMA((NBUF,)),
        pltpu.SemaphoreType.DMA((NBUF,)),
    ],
)
def _broadcast_rows(emb_hbm, out_hbm, buf, rsem, wsem):
    wid = lax.axis_index("s") * NC + lax.axis_index("c")
    base = wid * ROWS_PER_W

    read_h = [None] * NBUF
    write_h = [[] for _ in range(NBUF)]

    def start_read(ci):
        bi = ci % NBUF
        r0 = base + ci * CHUNK
        read_h[bi] = pltpu.async_copy(
            emb_hbm.at[pl.ds(r0, CHUNK)], buf.at[bi], rsem.at[bi]
        )

    start_read(0)
    for ci in range(NCHUNK):
        bi = ci % NBUF
        if ci + 1 < NCHUNK:
            nbi = (ci + 1) % NBUF
            for h in write_h[nbi]:
                h.wait()
            write_h[nbi] = []
            start_read(ci + 1)
        read_h[bi].wait()
        r0 = base + ci * CHUNK
        for b in range(B):
            write_h[bi].append(
                pltpu.async_copy(
                    buf.at[bi], out_hbm.at[b, pl.ds(r0, CHUNK)], wsem.at[bi]
                )
            )
    for bi in range(NBUF):
        for h in write_h[bi]:
            h.wait()


def kernel(x, embedding):
    del x
    return _broadcast_rows(embedding)


# speedup vs baseline: 1.0765x; 1.0765x over previous
import jax
import jax.numpy as jnp
from jax.experimental import pallas as pl

B, N, D = 4, 8192, 1024
GRID = 64
RB = N // GRID


def _body(emb_ref, out_ref):
    out_ref[...] = jnp.broadcast_to(emb_ref[...][None], (B, RB, D))


def kernel(x, embedding):
    del x
    return pl.pallas_call(
        _body,
        grid=(GRID,),
        in_specs=[pl.BlockSpec((RB, D), lambda i: (i, 0))],
        out_specs=pl.BlockSpec((B, RB, D), lambda i: (0, i, 0)),
        out_shape=jax.ShapeDtypeStruct((B, N, D), jnp.float32),
    )(embedding)
